# baseline (speedup 1.0000x reference)
import jax
import jax.numpy as jnp
from jax.experimental import pallas as pl
from jax.experimental.pallas import tpu as pltpu


def _fused_body(ph_ref, tok_ref, emb_ref, x_ref, wv_hbm, wo_hbm, bo_ref,
                wn_hbm, bn_ref, out_ref, wv_v, wo_v, wn_v, sv, so, sn):
    cv = pltpu.make_async_copy(wv_hbm, wv_v, sv)
    co = pltpu.make_async_copy(wo_hbm, wo_v, so)
    cn = pltpu.make_async_copy(wn_hbm, wn_v, sn)
    cv.start()
    co.start()
    cn.start()
    x = x_ref[...]
    cv.wait()
    t = jnp.dot(x, wv_v[...], preferred_element_type=jnp.float32)
    co.wait()
    t = jnp.dot(t, wo_v[...], preferred_element_type=jnp.float32) + bo_ref[...]
    cn.wait()
    p = jnp.dot(t, wn_v[...], preferred_element_type=jnp.float32) + bn_ref[...]
    mask = tok_ref[...] == ph_ref[0]
    out_ref[...] = jnp.where(mask, p, emb_ref[...])


def kernel(tokenized_text, embedded_text, image_embeds, placeholder_token,
           Wq1, Wk1, Wv1, Wo1, bo1, Wq2, Wk2, Wv2, Wo2, bo2, Wn, bn):
    b, n = tokenized_text.shape
    d = embedded_text.shape[-1]
    inner = Wv2.shape[-1]
    tok = tokenized_text.reshape(n, 1)
    emb = embedded_text.reshape(n, d)
    x = image_embeds.reshape(1, d)
    ph = placeholder_token.reshape(1)
    out = pl.pallas_call(
        _fused_body,
        out_shape=jax.ShapeDtypeStruct((n, d), jnp.float32),
        in_specs=[
            pl.BlockSpec(memory_space=pltpu.SMEM),
            pl.BlockSpec(memory_space=pltpu.VMEM),
            pl.BlockSpec(memory_space=pltpu.VMEM),
            pl.BlockSpec(memory_space=pltpu.VMEM),
            pl.BlockSpec(memory_space=pl.ANY),
            pl.BlockSpec(memory_space=pl.ANY),
            pl.BlockSpec(memory_space=pltpu.VMEM),
            pl.BlockSpec(memory_space=pl.ANY),
            pl.BlockSpec(memory_space=pltpu.VMEM),
        ],
        out_specs=pl.BlockSpec(memory_space=pltpu.VMEM),
        scratch_shapes=[
            pltpu.VMEM((d, inner), jnp.float32),
            pltpu.VMEM((inner, d), jnp.float32),
            pltpu.VMEM((d, d), jnp.float32),
            pltpu.SemaphoreType.DMA,
            pltpu.SemaphoreType.DMA,
            pltpu.SemaphoreType.DMA,
        ],
    )(ph, tok, emb, x, Wv2, Wo2, bo2.reshape(1, d), Wn, bn.reshape(1, d))
    return out.reshape(b, n, d)
